# Initial kernel scaffold; baseline (speedup 1.0000x reference)
import jax
import jax.numpy as jnp
from jax.experimental import pallas as pl


def kernel(x, edge_index, W1, b1, W2, b2):
    raise NotImplementedError("write your pallas kernel here")


# speedup vs baseline: 30.7374x; 30.7374x over previous
import functools

import jax
import jax.numpy as jnp
from jax import lax
from jax.experimental import pallas as pl
from jax.experimental.pallas import tpu as pltpu
from jax.experimental.pallas import tpu_sc as plsc

_N = 10000
_E = 320000
_D_IN = 128
_D_HID = 16
_NCLS = 7

_NC = 2
_NS = 16
_NW = _NC * _NS
_CH = 128
_NCHUNK = 80
_EPT = _CH * _NCHUNK
_E_PAD = _EPT * _NW
_N_PAD = 10240
_RPT = _N_PAD // _NS

_MESH = plsc.VectorSubcoreMesh(
    core_axis_name="c", subcore_axis_name="s", num_cores=_NC, num_subcores=_NS
)
_SC_PARAMS = pltpu.CompilerParams(use_tc_tiling_on_sc=False)


def _deg_kernel(dst_hbm, ones_hbm, zeros_hbm, out_hbm, dst_v, ones_v, acc_sh):
    c = lax.axis_index("c")
    s = lax.axis_index("s")
    pltpu.sync_copy(zeros_hbm, acc_sh.at[pl.ds(s * _RPT, _RPT)])
    pltpu.sync_copy(dst_hbm.at[c, s], dst_v)
    pltpu.sync_copy(ones_hbm, ones_v)
    plsc.subcore_barrier()

    def body(j, carry):
        pltpu.sync_copy(ones_v, acc_sh.at[dst_v.at[j]], add=True)
        return carry

    lax.fori_loop(0, _NCHUNK, body, 0)
    plsc.subcore_barrier()
    pltpu.sync_copy(
        acc_sh.at[pl.ds(s * _RPT, _RPT)], out_hbm.at[c, pl.ds(s * _RPT, _RPT)]
    )


def _make_deg_pass():
    return pl.kernel(
        _deg_kernel,
        out_type=jax.ShapeDtypeStruct((_NC, _N_PAD, 8), jnp.float32),
        mesh=_MESH,
        compiler_params=_SC_PARAMS,
        scratch_types=[
            pltpu.VMEM((_NCHUNK, _CH), jnp.int32),
            pltpu.VMEM((_CH, 8), jnp.float32),
            pltpu.VMEM_SHARED((_N_PAD, 8), jnp.float32),
        ],
    )


def _edge_kernel(
    table_hbm, src_hbm, dst_hbm, zeros_hbm, out_hbm, src_v, dst_v, rows_v, acc_sh, sem
):
    c = lax.axis_index("c")
    s = lax.axis_index("s")
    pltpu.sync_copy(zeros_hbm, acc_sh.at[pl.ds(s * _RPT, _RPT)])
    pltpu.sync_copy(src_hbm.at[c, s], src_v)
    pltpu.sync_copy(dst_hbm.at[c, s], dst_v)
    plsc.subcore_barrier()

    def body(j, carry):
        pltpu.async_copy(table_hbm.at[src_v.at[j]], rows_v, sem).wait()
        pltpu.sync_copy(rows_v, acc_sh.at[dst_v.at[j]], add=True)
        return carry

    lax.fori_loop(0, _NCHUNK, body, 0)
    plsc.subcore_barrier()
    pltpu.sync_copy(
        acc_sh.at[pl.ds(s * _RPT, _RPT)], out_hbm.at[c, pl.ds(s * _RPT, _RPT)]
    )


def _make_edge_pass(feat):
    return pl.kernel(
        _edge_kernel,
        out_type=jax.ShapeDtypeStruct((_NC, _N_PAD, feat), jnp.float32),
        mesh=_MESH,
        compiler_params=_SC_PARAMS,
        scratch_types=[
            pltpu.VMEM((_NCHUNK, _CH), jnp.int32),
            pltpu.VMEM((_NCHUNK, _CH), jnp.int32),
            pltpu.VMEM((_CH, feat), jnp.float32),
            pltpu.VMEM_SHARED((_N_PAD, feat), jnp.float32),
            pltpu.SemaphoreType.DMA,
        ],
    )


_BLK = 1024
_GRID = _N_PAD // _BLK


def _k1_body(x_ref, deg_ref, w1_ref, h1p_ref):
    dinv = lax.rsqrt(deg_ref[:, 0:1] + 1.0)
    h = jnp.dot(x_ref[...], w1_ref[...], preferred_element_type=jnp.float32)
    h1p_ref[...] = dinv * h


def _k2_body(s1_ref, h1p_ref, deg_ref, b1_ref, w2_ref, h2p_ref):
    dinv = lax.rsqrt(deg_ref[:, 0:1] + 1.0)
    tot = s1_ref[0] + s1_ref[1] + h1p_ref[...]
    out1 = jnp.maximum(dinv * tot + b1_ref[...], 0.0)
    h2p_ref[...] = dinv * jnp.dot(
        out1, w2_ref[...], preferred_element_type=jnp.float32
    )


def _k3_body(s2_ref, h2p_ref, deg_ref, b2_ref, out_ref):
    dinv = lax.rsqrt(deg_ref[:, 0:1] + 1.0)
    logits = dinv * (s2_ref[0] + s2_ref[1] + h2p_ref[...]) + b2_ref[...]
    col = lax.broadcasted_iota(jnp.int32, logits.shape, 1)
    valid = col < _NCLS
    neg = jnp.float32(-jnp.inf)
    masked = jnp.where(valid, logits, neg)
    m = jnp.max(masked, axis=1, keepdims=True)
    e = jnp.where(valid, jnp.exp(logits - m), 0.0)
    lse = jnp.log(jnp.sum(e, axis=1, keepdims=True))
    out_ref[...] = logits - m - lse


def _row_spec(feat):
    return pl.BlockSpec((_BLK, feat), lambda i: (i, 0))


def _pair_spec(feat):
    return pl.BlockSpec((_NC, _BLK, feat), lambda i: (0, i, 0))


def _full_spec(shape):
    return pl.BlockSpec(shape, lambda i: tuple(0 for _ in shape))


def kernel(x, edge_index, W1, b1, W2, b2):
    src = edge_index[0]
    dst = edge_index[1]
    pad_idx = jnp.full((_E_PAD - _E,), _N, dtype=jnp.int32)
    src_p = jnp.concatenate([src, pad_idx]).reshape(_NC, _NS, _NCHUNK, _CH)
    dst_p = jnp.concatenate([dst, pad_idx]).reshape(_NC, _NS, _NCHUNK, _CH)

    x_p = jnp.zeros((_N_PAD, _D_IN), jnp.float32).at[:_N].set(x)
    w2_p = jnp.zeros((_D_HID, 8), jnp.float32).at[:, :_NCLS].set(W2)
    b1_r = b1.reshape(1, _D_HID)
    b2_p = jnp.zeros((1, 8), jnp.float32).at[0, :_NCLS].set(b2)

    ones_rows = jnp.ones((_CH, 8), jnp.float32)
    zeros16 = jnp.zeros((_RPT, _D_HID), jnp.float32)
    zeros8 = jnp.zeros((_RPT, 8), jnp.float32)

    _DBG_JNP_DEG = False
    if _DBG_JNP_DEG:
        deg1 = jnp.zeros((_N_PAD,), jnp.float32).at[dst_p.reshape(-1)].add(1.0)
        deg = jnp.broadcast_to(deg1[:, None], (_N_PAD, 8))
    else:
        deg8 = _make_deg_pass()(dst_p, ones_rows, zeros8)
        deg = deg8[0] + deg8[1]

    h1p = pl.pallas_call(
        _k1_body,
        grid=(_GRID,),
        in_specs=[
            _row_spec(_D_IN),
            _row_spec(8),
            _full_spec((_D_IN, _D_HID)),
        ],
        out_specs=_row_spec(_D_HID),
        out_shape=jax.ShapeDtypeStruct((_N_PAD, _D_HID), jnp.float32),
    )(x_p, deg, W1)

    _DBG_JNP_EDGE = False
    if _DBG_JNP_EDGE:
        sflat = src_p.reshape(-1)
        dflat = dst_p.reshape(-1)
        s1_dbg = jnp.zeros((_N_PAD, _D_HID), jnp.float32).at[dflat].add(h1p[sflat])
        s1 = jnp.stack([s1_dbg, jnp.zeros_like(s1_dbg)])
    else:
        s1 = _make_edge_pass(_D_HID)(h1p, src_p, dst_p, zeros16)

    h2p = pl.pallas_call(
        _k2_body,
        grid=(_GRID,),
        in_specs=[
            _pair_spec(_D_HID),
            _row_spec(_D_HID),
            _row_spec(8),
            _full_spec((1, _D_HID)),
            _full_spec((_D_HID, 8)),
        ],
        out_specs=_row_spec(8),
        out_shape=jax.ShapeDtypeStruct((_N_PAD, 8), jnp.float32),
    )(s1, h1p, deg, b1_r, w2_p)

    if _DBG_JNP_EDGE:
        s2_dbg = jnp.zeros((_N_PAD, 8), jnp.float32).at[dflat].add(h2p[sflat])
        s2 = jnp.stack([s2_dbg, jnp.zeros_like(s2_dbg)])
    else:
        s2 = _make_edge_pass(8)(h2p, src_p, dst_p, zeros8)

    out8 = pl.pallas_call(
        _k3_body,
        grid=(_GRID,),
        in_specs=[
            _pair_spec(8),
            _row_spec(8),
            _row_spec(8),
            _full_spec((1, 8)),
        ],
        out_specs=_row_spec(8),
        out_shape=jax.ShapeDtypeStruct((_N_PAD, 8), jnp.float32),
    )(s2, h2p, deg, b2_p)

    return out8[:_N, :_NCLS]


# speedup vs baseline: 39.1930x; 1.2751x over previous
import functools

import jax
import jax.numpy as jnp
from jax import lax
from jax.experimental import pallas as pl
from jax.experimental.pallas import tpu as pltpu
from jax.experimental.pallas import tpu_sc as plsc

_N = 10000
_E = 320000
_D_IN = 128
_D_HID = 16
_NCLS = 7

_NC = 2
_NS = 16
_NW = _NC * _NS
_CH = 128
_NCHUNK = 80
_EPT = _CH * _NCHUNK
_E_PAD = _EPT * _NW
_N_PAD = 10240
_RPT = _N_PAD // _NS

_MESH = plsc.VectorSubcoreMesh(
    core_axis_name="c", subcore_axis_name="s", num_cores=_NC, num_subcores=_NS
)
_SC_PARAMS = pltpu.CompilerParams(use_tc_tiling_on_sc=False)


def _deg_kernel(dst_hbm, ones_hbm, zeros_hbm, out_hbm, dst_v, ones_v, acc_sh):
    c = lax.axis_index("c")
    s = lax.axis_index("s")
    pltpu.sync_copy(zeros_hbm, acc_sh.at[pl.ds(s * _RPT, _RPT)])
    pltpu.sync_copy(dst_hbm.at[c, s], dst_v)
    pltpu.sync_copy(ones_hbm, ones_v)
    plsc.subcore_barrier()

    def body(j, carry):
        pltpu.sync_copy(ones_v, acc_sh.at[dst_v.at[j]], add=True)
        return carry

    lax.fori_loop(0, _NCHUNK, body, 0)
    plsc.subcore_barrier()
    pltpu.sync_copy(
        acc_sh.at[pl.ds(s * _RPT, _RPT)], out_hbm.at[c, pl.ds(s * _RPT, _RPT)]
    )


def _make_deg_pass():
    return pl.kernel(
        _deg_kernel,
        out_type=jax.ShapeDtypeStruct((_NC, _N_PAD, 8), jnp.float32),
        mesh=_MESH,
        compiler_params=_SC_PARAMS,
        scratch_types=[
            pltpu.VMEM((_NCHUNK, _CH), jnp.int32),
            pltpu.VMEM((_CH, 8), jnp.float32),
            pltpu.VMEM_SHARED((_N_PAD, 8), jnp.float32),
        ],
    )


def _edge_kernel(
    table_hbm,
    src_hbm,
    dst_hbm,
    zeros_hbm,
    out_hbm,
    src_v,
    dst_v,
    buf0,
    buf1,
    acc_sh,
    sem0,
    sem1,
):
    c = lax.axis_index("c")
    s = lax.axis_index("s")
    pltpu.sync_copy(zeros_hbm, acc_sh.at[pl.ds(s * _RPT, _RPT)])
    pltpu.sync_copy(src_hbm.at[c, s], src_v)
    pltpu.sync_copy(dst_hbm.at[c, s], dst_v)
    plsc.subcore_barrier()

    pltpu.async_copy(table_hbm.at[src_v.at[0]], buf0, sem0)

    def body(j2, carry):
        j = 2 * j2
        pltpu.async_copy(table_hbm.at[src_v.at[j + 1]], buf1, sem1)
        pltpu.make_async_copy(table_hbm.at[src_v.at[j]], buf0, sem0).wait()
        pltpu.sync_copy(buf0, acc_sh.at[dst_v.at[j]], add=True)

        @pl.when(j2 < _NCHUNK // 2 - 1)
        def _():
            pltpu.async_copy(table_hbm.at[src_v.at[j + 2]], buf0, sem0)

        pltpu.make_async_copy(table_hbm.at[src_v.at[j + 1]], buf1, sem1).wait()
        pltpu.sync_copy(buf1, acc_sh.at[dst_v.at[j + 1]], add=True)
        return carry

    lax.fori_loop(0, _NCHUNK // 2, body, 0)
    plsc.subcore_barrier()
    pltpu.sync_copy(
        acc_sh.at[pl.ds(s * _RPT, _RPT)], out_hbm.at[c, pl.ds(s * _RPT, _RPT)]
    )


def _make_edge_pass(feat):
    return pl.kernel(
        _edge_kernel,
        out_type=jax.ShapeDtypeStruct((_NC, _N_PAD, feat), jnp.float32),
        mesh=_MESH,
        compiler_params=_SC_PARAMS,
        scratch_types=[
            pltpu.VMEM((_NCHUNK, _CH), jnp.int32),
            pltpu.VMEM((_NCHUNK, _CH), jnp.int32),
            pltpu.VMEM((_CH, feat), jnp.float32),
            pltpu.VMEM((_CH, feat), jnp.float32),
            pltpu.VMEM_SHARED((_N_PAD, feat), jnp.float32),
            pltpu.SemaphoreType.DMA,
            pltpu.SemaphoreType.DMA,
        ],
    )


_BLK = 1024
_GRID = _N_PAD // _BLK


def _k1_body(x_ref, deg_ref, w1_ref, h1p_ref):
    dinv = lax.rsqrt(deg_ref[:, 0:1] + 1.0)
    h = jnp.dot(x_ref[...], w1_ref[...], preferred_element_type=jnp.float32)
    h1p_ref[...] = dinv * h


def _k2_body(s1_ref, h1p_ref, deg_ref, b1_ref, w2_ref, h2p_ref):
    dinv = lax.rsqrt(deg_ref[:, 0:1] + 1.0)
    tot = s1_ref[0] + s1_ref[1] + h1p_ref[...]
    out1 = jnp.maximum(dinv * tot + b1_ref[...], 0.0)
    h2p_ref[...] = dinv * jnp.dot(
        out1, w2_ref[...], preferred_element_type=jnp.float32
    )


def _k3_body(s2_ref, h2p_ref, deg_ref, b2_ref, out_ref):
    dinv = lax.rsqrt(deg_ref[:, 0:1] + 1.0)
    logits = dinv * (s2_ref[0] + s2_ref[1] + h2p_ref[...]) + b2_ref[...]
    col = lax.broadcasted_iota(jnp.int32, logits.shape, 1)
    valid = col < _NCLS
    neg = jnp.float32(-jnp.inf)
    masked = jnp.where(valid, logits, neg)
    m = jnp.max(masked, axis=1, keepdims=True)
    e = jnp.where(valid, jnp.exp(logits - m), 0.0)
    lse = jnp.log(jnp.sum(e, axis=1, keepdims=True))
    out_ref[...] = logits - m - lse


def _row_spec(feat):
    return pl.BlockSpec((_BLK, feat), lambda i: (i, 0))


def _pair_spec(feat):
    return pl.BlockSpec((_NC, _BLK, feat), lambda i: (0, i, 0))


def _full_spec(shape):
    return pl.BlockSpec(shape, lambda i: tuple(0 for _ in shape))


def kernel(x, edge_index, W1, b1, W2, b2):
    src = edge_index[0]
    dst = edge_index[1]
    pad_idx = jnp.full((_E_PAD - _E,), _N, dtype=jnp.int32)
    src_p = jnp.concatenate([src, pad_idx]).reshape(_NC, _NS, _NCHUNK, _CH)
    dst_p = jnp.concatenate([dst, pad_idx]).reshape(_NC, _NS, _NCHUNK, _CH)

    x_p = jnp.zeros((_N_PAD, _D_IN), jnp.float32).at[:_N].set(x)
    w2_p = jnp.zeros((_D_HID, 8), jnp.float32).at[:, :_NCLS].set(W2)
    b1_r = b1.reshape(1, _D_HID)
    b2_p = jnp.zeros((1, 8), jnp.float32).at[0, :_NCLS].set(b2)

    ones_rows = jnp.ones((_CH, 8), jnp.float32)
    zeros16 = jnp.zeros((_RPT, _D_HID), jnp.float32)
    zeros8 = jnp.zeros((_RPT, 8), jnp.float32)

    _DBG_JNP_DEG = False
    if _DBG_JNP_DEG:
        deg1 = jnp.zeros((_N_PAD,), jnp.float32).at[dst_p.reshape(-1)].add(1.0)
        deg = jnp.broadcast_to(deg1[:, None], (_N_PAD, 8))
    else:
        deg8 = _make_deg_pass()(dst_p, ones_rows, zeros8)
        deg = deg8[0] + deg8[1]

    h1p = pl.pallas_call(
        _k1_body,
        grid=(_GRID,),
        in_specs=[
            _row_spec(_D_IN),
            _row_spec(8),
            _full_spec((_D_IN, _D_HID)),
        ],
        out_specs=_row_spec(_D_HID),
        out_shape=jax.ShapeDtypeStruct((_N_PAD, _D_HID), jnp.float32),
    )(x_p, deg, W1)

    _DBG_JNP_EDGE = False
    if _DBG_JNP_EDGE:
        sflat = src_p.reshape(-1)
        dflat = dst_p.reshape(-1)
        s1_dbg = jnp.zeros((_N_PAD, _D_HID), jnp.float32).at[dflat].add(h1p[sflat])
        s1 = jnp.stack([s1_dbg, jnp.zeros_like(s1_dbg)])
    else:
        s1 = _make_edge_pass(_D_HID)(h1p, src_p, dst_p, zeros16)

    h2p = pl.pallas_call(
        _k2_body,
        grid=(_GRID,),
        in_specs=[
            _pair_spec(_D_HID),
            _row_spec(_D_HID),
            _row_spec(8),
            _full_spec((1, _D_HID)),
            _full_spec((_D_HID, 8)),
        ],
        out_specs=_row_spec(8),
        out_shape=jax.ShapeDtypeStruct((_N_PAD, 8), jnp.float32),
    )(s1, h1p, deg, b1_r, w2_p)

    if _DBG_JNP_EDGE:
        s2_dbg = jnp.zeros((_N_PAD, 8), jnp.float32).at[dflat].add(h2p[sflat])
        s2 = jnp.stack([s2_dbg, jnp.zeros_like(s2_dbg)])
    else:
        s2 = _make_edge_pass(8)(h2p, src_p, dst_p, zeros8)

    out8 = pl.pallas_call(
        _k3_body,
        grid=(_GRID,),
        in_specs=[
            _pair_spec(8),
            _row_spec(8),
            _row_spec(8),
            _full_spec((1, 8)),
        ],
        out_specs=_row_spec(8),
        out_shape=jax.ShapeDtypeStruct((_N_PAD, 8), jnp.float32),
    )(s2, h2p, deg, b2_p)

    return out8[:_N, :_NCLS]


# speedup vs baseline: 45.5634x; 1.1625x over previous
import functools

import jax
import jax.numpy as jnp
from jax import lax
from jax.experimental import pallas as pl
from jax.experimental.pallas import tpu as pltpu
from jax.experimental.pallas import tpu_sc as plsc

_N = 10000
_E = 320000
_D_IN = 128
_D_HID = 16
_NCLS = 7

_NC = 2
_NS = 16
_NW = _NC * _NS
_NCHUNK = 80
_CH = _E // _NW // _NCHUNK
_RPT = _N // _NS

_MESH = plsc.VectorSubcoreMesh(
    core_axis_name="c", subcore_axis_name="s", num_cores=_NC, num_subcores=_NS
)
_SC_PARAMS = pltpu.CompilerParams(use_tc_tiling_on_sc=False)


def _deg_kernel(dst_hbm, ones_hbm, zeros_hbm, out_hbm, dst_v, ones_v, acc_sh):
    c = lax.axis_index("c")
    s = lax.axis_index("s")
    pltpu.sync_copy(zeros_hbm, acc_sh.at[pl.ds(s * _RPT, _RPT)])
    pltpu.sync_copy(dst_hbm.at[c, s], dst_v)
    pltpu.sync_copy(ones_hbm, ones_v)
    plsc.subcore_barrier()

    def body(j, carry):
        pltpu.sync_copy(ones_v, acc_sh.at[dst_v.at[j]], add=True)
        return carry

    lax.fori_loop(0, _NCHUNK, body, 0)
    plsc.subcore_barrier()
    pltpu.sync_copy(
        acc_sh.at[pl.ds(s * _RPT, _RPT)], out_hbm.at[c, pl.ds(s * _RPT, _RPT)]
    )


def _make_deg_pass():
    return pl.kernel(
        _deg_kernel,
        out_type=jax.ShapeDtypeStruct((_NC, _N, 8), jnp.float32),
        mesh=_MESH,
        compiler_params=_SC_PARAMS,
        scratch_types=[
            pltpu.VMEM((_NCHUNK, _CH), jnp.int32),
            pltpu.VMEM((_CH, 8), jnp.float32),
            pltpu.VMEM_SHARED((_N, 8), jnp.float32),
        ],
    )


def _edge_kernel(
    table_hbm,
    src_hbm,
    dst_hbm,
    zeros_hbm,
    out_hbm,
    src_v,
    dst_v,
    buf0,
    buf1,
    acc_sh,
    sem0,
    sem1,
):
    c = lax.axis_index("c")
    s = lax.axis_index("s")
    pltpu.sync_copy(zeros_hbm, acc_sh.at[pl.ds(s * _RPT, _RPT)])
    pltpu.sync_copy(src_hbm.at[c, s], src_v)
    pltpu.sync_copy(dst_hbm.at[c, s], dst_v)
    plsc.subcore_barrier()

    pltpu.async_copy(table_hbm.at[src_v.at[0]], buf0, sem0)

    def body(j2, carry):
        j = 2 * j2
        pltpu.async_copy(table_hbm.at[src_v.at[j + 1]], buf1, sem1)
        pltpu.make_async_copy(table_hbm.at[src_v.at[j]], buf0, sem0).wait()
        pltpu.sync_copy(buf0, acc_sh.at[dst_v.at[j]], add=True)

        @pl.when(j2 < _NCHUNK // 2 - 1)
        def _():
            pltpu.async_copy(table_hbm.at[src_v.at[j + 2]], buf0, sem0)

        pltpu.make_async_copy(table_hbm.at[src_v.at[j + 1]], buf1, sem1).wait()
        pltpu.sync_copy(buf1, acc_sh.at[dst_v.at[j + 1]], add=True)
        return carry

    lax.fori_loop(0, _NCHUNK // 2, body, 0)
    plsc.subcore_barrier()
    pltpu.sync_copy(
        acc_sh.at[pl.ds(s * _RPT, _RPT)], out_hbm.at[c, pl.ds(s * _RPT, _RPT)]
    )


def _make_edge_pass(feat):
    return pl.kernel(
        _edge_kernel,
        out_type=jax.ShapeDtypeStruct((_NC, _N, feat), jnp.float32),
        mesh=_MESH,
        compiler_params=_SC_PARAMS,
        scratch_types=[
            pltpu.VMEM((_NCHUNK, _CH), jnp.int32),
            pltpu.VMEM((_NCHUNK, _CH), jnp.int32),
            pltpu.VMEM((_CH, feat), jnp.float32),
            pltpu.VMEM((_CH, feat), jnp.float32),
            pltpu.VMEM_SHARED((_N, feat), jnp.float32),
            pltpu.SemaphoreType.DMA,
            pltpu.SemaphoreType.DMA,
        ],
    )


_BLK = 1000
_GRID = _N // _BLK


def _k1_body(x_ref, deg_ref, w1_ref, h1p_ref):
    deg = deg_ref[0, :, 0:1] + deg_ref[1, :, 0:1]
    dinv = lax.rsqrt(deg + 1.0)
    h = jnp.dot(x_ref[...], w1_ref[...], preferred_element_type=jnp.float32)
    h1p_ref[...] = dinv * h


def _k2_body(s1_ref, h1p_ref, deg_ref, b1_ref, w2_ref, h2p_ref):
    deg = deg_ref[0, :, 0:1] + deg_ref[1, :, 0:1]
    dinv = lax.rsqrt(deg + 1.0)
    tot = s1_ref[0] + s1_ref[1] + h1p_ref[...]
    out1 = jnp.maximum(dinv * tot + b1_ref[...], 0.0)
    h2p_ref[...] = dinv * jnp.dot(
        out1, w2_ref[...], preferred_element_type=jnp.float32
    )


def _k3_body(s2_ref, h2p_ref, deg_ref, b2_ref, out_ref):
    deg = deg_ref[0, :, 0:1] + deg_ref[1, :, 0:1]
    dinv = lax.rsqrt(deg + 1.0)
    logits = dinv * (s2_ref[0] + s2_ref[1] + h2p_ref[...]) + b2_ref[...]
    col = lax.broadcasted_iota(jnp.int32, logits.shape, 1)
    valid = col < _NCLS
    neg = jnp.float32(-jnp.inf)
    masked = jnp.where(valid, logits, neg)
    m = jnp.max(masked, axis=1, keepdims=True)
    e = jnp.where(valid, jnp.exp(logits - m), 0.0)
    lse = jnp.log(jnp.sum(e, axis=1, keepdims=True))
    out_ref[...] = (logits - m - lse)[:, :_NCLS]


def _row_spec(feat):
    return pl.BlockSpec((_BLK, feat), lambda i: (i, 0))


def _pair_spec(feat):
    return pl.BlockSpec((_NC, _BLK, feat), lambda i: (0, i, 0))


def _full_spec(shape):
    return pl.BlockSpec(shape, lambda i: tuple(0 for _ in shape))


def kernel(x, edge_index, W1, b1, W2, b2):
    src_p = edge_index[0].reshape(_NC, _NS, _NCHUNK, _CH)
    dst_p = edge_index[1].reshape(_NC, _NS, _NCHUNK, _CH)

    w2_p = jnp.zeros((_D_HID, 8), jnp.float32).at[:, :_NCLS].set(W2)
    b1_r = b1.reshape(1, _D_HID)
    b2_p = jnp.zeros((1, 8), jnp.float32).at[0, :_NCLS].set(b2)

    ones_rows = jnp.ones((_CH, 8), jnp.float32)
    zeros16 = jnp.zeros((_RPT, _D_HID), jnp.float32)
    zeros8 = jnp.zeros((_RPT, 8), jnp.float32)

    deg8 = _make_deg_pass()(dst_p, ones_rows, zeros8)

    h1p = pl.pallas_call(
        _k1_body,
        grid=(_GRID,),
        in_specs=[
            _row_spec(_D_IN),
            _pair_spec(8),
            _full_spec((_D_IN, _D_HID)),
        ],
        out_specs=_row_spec(_D_HID),
        out_shape=jax.ShapeDtypeStruct((_N, _D_HID), jnp.float32),
    )(x, deg8, W1)

    s1 = _make_edge_pass(_D_HID)(h1p, src_p, dst_p, zeros16)

    h2p = pl.pallas_call(
        _k2_body,
        grid=(_GRID,),
        in_specs=[
            _pair_spec(_D_HID),
            _row_spec(_D_HID),
            _pair_spec(8),
            _full_spec((1, _D_HID)),
            _full_spec((_D_HID, 8)),
        ],
        out_specs=_row_spec(8),
        out_shape=jax.ShapeDtypeStruct((_N, 8), jnp.float32),
    )(s1, h1p, deg8, b1_r, w2_p)

    s2 = _make_edge_pass(8)(h2p, src_p, dst_p, zeros8)

    out = pl.pallas_call(
        _k3_body,
        grid=(_GRID,),
        in_specs=[
            _pair_spec(8),
            _row_spec(8),
            _pair_spec(8),
            _full_spec((1, 8)),
        ],
        out_specs=pl.BlockSpec((_BLK, _NCLS), lambda i: (i, 0)),
        out_shape=jax.ShapeDtypeStruct((_N, _NCLS), jnp.float32),
    )(s2, h2p, deg8, b2_p)

    return out


# speedup vs baseline: 78.9643x; 1.7331x over previous
import functools

import jax
import jax.numpy as jnp
from jax import lax
from jax.experimental import pallas as pl
from jax.experimental.pallas import tpu as pltpu
from jax.experimental.pallas import tpu_sc as plsc

_N = 10000
_E = 320000
_D_IN = 128
_D_HID = 16
_NCLS = 7

_NP = 10048
_PR = _NP * 16 // 128

_NC = 2
_NS = 16
_NW = _NC * _NS
_NCHUNK = 80
_CH = _E // _NW // _NCHUNK
_RPT = _NP // _NS

_MESH = plsc.VectorSubcoreMesh(
    core_axis_name="c", subcore_axis_name="s", num_cores=_NC, num_subcores=_NS
)
_SC_PARAMS = pltpu.CompilerParams(use_tc_tiling_on_sc=False)


def _deg_kernel(edges_hbm, ones_hbm, zeros_hbm, out_hbm, dst_v, ones_v, acc_sh, sem):
    c = lax.axis_index("c")
    s = lax.axis_index("s")
    pltpu.sync_copy(zeros_hbm, acc_sh.at[pl.ds(s * _RPT, _RPT)])
    pltpu.sync_copy(edges_hbm.at[1, c, s], dst_v)
    pltpu.sync_copy(ones_hbm, ones_v)
    plsc.subcore_barrier()

    def body(j, carry):
        pltpu.async_copy(ones_v, acc_sh.at[dst_v.at[j]], sem, add=True)
        return carry

    lax.fori_loop(0, _NCHUNK, body, 0)

    def drain(j, carry):
        pltpu.make_async_copy(ones_v, acc_sh.at[dst_v.at[0]], sem).wait()
        return carry

    lax.fori_loop(0, _NCHUNK, drain, 0)
    plsc.subcore_barrier()
    pltpu.sync_copy(
        acc_sh.at[pl.ds(s * _RPT, _RPT)], out_hbm.at[c, pl.ds(s * _RPT, _RPT)]
    )


def _make_deg_pass():
    return pl.kernel(
        _deg_kernel,
        out_type=jax.ShapeDtypeStruct((_NC, _NP, 16), jnp.float32),
        mesh=_MESH,
        compiler_params=_SC_PARAMS,
        scratch_types=[
            pltpu.VMEM((_NCHUNK, _CH), jnp.int32),
            pltpu.VMEM((_CH, 16), jnp.float32),
            pltpu.VMEM_SHARED((_NP, 16), jnp.float32),
            pltpu.SemaphoreType.DMA,
        ],
    )


def _edge_kernel(
    table_hbm,
    edges_hbm,
    zeros_hbm,
    out_hbm,
    src_v,
    dst_v,
    buf0,
    buf1,
    buf2,
    buf3,
    acc_sh,
    gsem0,
    gsem1,
    gsem2,
    gsem3,
    ssem0,
    ssem1,
    ssem2,
    ssem3,
):
    c = lax.axis_index("c")
    s = lax.axis_index("s")
    pltpu.sync_copy(zeros_hbm, acc_sh.at[pl.ds(s * _RPT, _RPT)])
    pltpu.sync_copy(edges_hbm.at[0, c, s], src_v)
    pltpu.sync_copy(edges_hbm.at[1, c, s], dst_v)
    plsc.subcore_barrier()

    bufs = [buf0, buf1, buf2, buf3]
    gsems = [gsem0, gsem1, gsem2, gsem3]
    ssems = [ssem0, ssem1, ssem2, ssem3]

    for b in range(3):
        pltpu.async_copy(table_hbm.at[src_v.at[b]], bufs[b], gsems[b])

    def body(g, carry):
        for b in range(4):
            j = 4 * g + b
            bn = (b + 3) % 4
            jn = j + 3
            pltpu.make_async_copy(table_hbm.at[src_v.at[j]], bufs[b], gsems[b]).wait()
            pltpu.async_copy(bufs[b], acc_sh.at[dst_v.at[j]], ssems[b], add=True)
            if b == 0:

                @pl.when(g > 0)
                def _():
                    pltpu.make_async_copy(
                        bufs[bn], acc_sh.at[dst_v.at[0]], ssems[bn]
                    ).wait()

                pltpu.async_copy(table_hbm.at[src_v.at[jn]], bufs[bn], gsems[bn])
            else:

                @pl.when(g < _NCHUNK // 4 - 1)
                def _():
                    pltpu.make_async_copy(
                        bufs[bn], acc_sh.at[dst_v.at[0]], ssems[bn]
                    ).wait()
                    pltpu.async_copy(table_hbm.at[src_v.at[jn]], bufs[bn], gsems[bn])

        return carry

    lax.fori_loop(0, _NCHUNK // 4, body, 0)
    for b in range(4):
        pltpu.make_async_copy(bufs[b], acc_sh.at[dst_v.at[0]], ssems[b]).wait()
    plsc.subcore_barrier()
    pltpu.sync_copy(
        acc_sh.at[pl.ds(s * _RPT, _RPT)], out_hbm.at[c, pl.ds(s * _RPT, _RPT)]
    )


def _make_edge_pass():
    return pl.kernel(
        _edge_kernel,
        out_type=jax.ShapeDtypeStruct((_NC, _NP, 16), jnp.float32),
        mesh=_MESH,
        compiler_params=_SC_PARAMS,
        scratch_types=[
            pltpu.VMEM((_NCHUNK, _CH), jnp.int32),
            pltpu.VMEM((_NCHUNK, _CH), jnp.int32),
            pltpu.VMEM((_CH, 16), jnp.float32),
            pltpu.VMEM((_CH, 16), jnp.float32),
            pltpu.VMEM((_CH, 16), jnp.float32),
            pltpu.VMEM((_CH, 16), jnp.float32),
            pltpu.VMEM_SHARED((_NP, 16), jnp.float32),
            pltpu.SemaphoreType.DMA,
            pltpu.SemaphoreType.DMA,
            pltpu.SemaphoreType.DMA,
            pltpu.SemaphoreType.DMA,
            pltpu.SemaphoreType.DMA,
            pltpu.SemaphoreType.DMA,
            pltpu.SemaphoreType.DMA,
            pltpu.SemaphoreType.DMA,
        ],
    )


def _dinv_packed(deg_ref):
    return lax.rsqrt(deg_ref[0] + deg_ref[1] + 1.0)


def _k1_body(xp_ref, deg_ref, bd1_ref, h1p_ref):
    h = jnp.dot(xp_ref[...], bd1_ref[...], preferred_element_type=jnp.float32)
    h1p_ref[...] = _dinv_packed(deg_ref) * h


def _k2_body(s1_ref, h1p_ref, deg_ref, b1_ref, bd2_ref, h2p_ref):
    dinv = _dinv_packed(deg_ref)
    out1 = jnp.maximum(dinv * (s1_ref[0] + s1_ref[1] + h1p_ref[...]) + b1_ref[...], 0.0)
    h2p_ref[...] = dinv * jnp.dot(
        out1, bd2_ref[...], preferred_element_type=jnp.float32
    )


def _k3_body(s2_ref, h2p_ref, deg_ref, b2_ref, gm_ref, gs_ref, out_ref):
    dinv = _dinv_packed(deg_ref)
    logits = dinv * (s2_ref[0] + s2_ref[1] + h2p_ref[...]) + b2_ref[...]
    mu = jnp.dot(logits, gm_ref[...], preferred_element_type=jnp.float32)
    z = logits - mu
    e = jnp.exp(z)
    ssum = jnp.dot(e, gs_ref[...], preferred_element_type=jnp.float32)
    out_ref[...] = z - jnp.log(ssum)


def _full_spec(shape):
    return pl.BlockSpec(shape, lambda: tuple(0 for _ in shape))


def kernel(x, edge_index, W1, b1, W2, b2):
    edges5 = edge_index.reshape(2, _NC, _NS, _NCHUNK, _CH)

    w2_p = jnp.zeros((_D_HID, 16), jnp.float32).at[:, :_NCLS].set(W2)
    bd2 = jnp.kron(jnp.eye(8, dtype=jnp.float32), w2_p)
    b1_row = jnp.tile(b1, 8).reshape(1, 128)
    b2_row = jnp.tile(
        jnp.zeros((16,), jnp.float32).at[:_NCLS].set(b2), 8
    ).reshape(1, 128)
    vblk = jnp.zeros((16, 16), jnp.float32).at[:_NCLS, :].set(1.0)
    gs = jnp.kron(jnp.eye(8, dtype=jnp.float32), vblk)
    gm = gs / jnp.float32(_NCLS)

    ones_rows = jnp.ones((_CH, 16), jnp.float32)
    zeros16 = jnp.zeros((_RPT, 16), jnp.float32)

    deg16 = _make_deg_pass()(edges5, ones_rows, zeros16)
    deg_p = deg16.reshape(_NC, _PR, 128)

    bd1 = jnp.kron(jnp.eye(8, dtype=jnp.float32), W1)
    x_p = jnp.zeros((_PR, 8 * _D_IN), jnp.float32).at[: _N // 8].set(
        x.reshape(_N // 8, 8 * _D_IN)
    )

    h1p = pl.pallas_call(
        _k1_body,
        in_specs=[
            _full_spec((_PR, 8 * _D_IN)),
            _full_spec((_NC, _PR, 128)),
            _full_spec((8 * _D_IN, 128)),
        ],
        out_specs=_full_spec((_PR, 128)),
        out_shape=jax.ShapeDtypeStruct((_PR, 128), jnp.float32),
    )(x_p, deg_p, bd1)

    s1 = _make_edge_pass()(h1p.reshape(_NP, 16), edges5, zeros16)

    h2p = pl.pallas_call(
        _k2_body,
        in_specs=[
            _full_spec((_NC, _PR, 128)),
            _full_spec((_PR, 128)),
            _full_spec((_NC, _PR, 128)),
            _full_spec((1, 128)),
            _full_spec((128, 128)),
        ],
        out_specs=_full_spec((_PR, 128)),
        out_shape=jax.ShapeDtypeStruct((_PR, 128), jnp.float32),
    )(s1.reshape(_NC, _PR, 128), h1p, deg_p, b1_row, bd2)

    s2 = _make_edge_pass()(h2p.reshape(_NP, 16), edges5, zeros16)

    outp = pl.pallas_call(
        _k3_body,
        in_specs=[
            _full_spec((_NC, _PR, 128)),
            _full_spec((_PR, 128)),
            _full_spec((_NC, _PR, 128)),
            _full_spec((1, 128)),
            _full_spec((128, 128)),
            _full_spec((128, 128)),
        ],
        out_specs=_full_spec((_PR, 128)),
        out_shape=jax.ShapeDtypeStruct((_PR, 128), jnp.float32),
    )(s2.reshape(_NC, _PR, 128), h2p, deg_p, b2_row, gm, gs)

    return outp.reshape(_NP, 16)[:_N, :_NCLS]
